# baseline (speedup 1.0000x reference)
import functools

import jax
import jax.numpy as jnp
from jax import lax
from jax.experimental import pallas as pl
from jax.experimental.pallas import tpu as pltpu
from jax.experimental.pallas import tpu_sc as plsc

VOCAB = 100000
HID = 768
MAXPOS = 1024
EPS = 1e-12

NC = 2
NS = 16
NW = NC * NS

B = 32
NSTAGE = 4
BH = B // NSTAGE
LT = 512
LF = 512
SEQ = LT + LF
NTOK_H = BH * LT
ROWS_PER_W = NTOK_H // NW
CHUNK = 64
NCHUNK = ROWS_PER_W // CHUNK


def _sc_gather_body(table_hbm, idx_hbm, out_hbm, idx_v, buf0, buf1,
                    sg0, sg1, sw0, sw1):
    wid = lax.axis_index("s") * NC + lax.axis_index("c")
    base = wid * ROWS_PER_W
    pltpu.sync_copy(idx_hbm.at[pl.ds(base, ROWS_PER_W)], idx_v)
    bufs = (buf0, buf1)
    gsems = (sg0, sg1)
    wsems = (sw0, sw1)
    gathers = [None, None]
    writes = [None, None]
    for c in range(NCHUNK):
        p = c % 2
        if writes[p] is not None:
            writes[p].wait()
        gathers[p] = pltpu.async_copy(
            table_hbm.at[idx_v.at[pl.ds(c * CHUNK, CHUNK)]], bufs[p], gsems[p]
        )
        if c >= 1:
            q = (c - 1) % 2
            gathers[q].wait()
            writes[q] = pltpu.async_copy(
                bufs[q], out_hbm.at[pl.ds(base + (c - 1) * CHUNK, CHUNK)],
                wsems[q],
            )
    p = (NCHUNK - 1) % 2
    gathers[p].wait()
    writes[p] = pltpu.async_copy(
        bufs[p], out_hbm.at[pl.ds(base + (NCHUNK - 1) * CHUNK, CHUNK)], wsems[p]
    )
    writes[p].wait()
    writes[1 - p].wait()


@functools.cache
def _make_sc_gather():
    return pl.kernel(
        _sc_gather_body,
        mesh=plsc.VectorSubcoreMesh(
            core_axis_name="c", subcore_axis_name="s",
            num_cores=NC, num_subcores=NS,
        ),
        out_type=jax.ShapeDtypeStruct((NTOK_H, HID), jnp.float32),
        scratch_types=[
            pltpu.VMEM((ROWS_PER_W,), jnp.int32),
            pltpu.VMEM((CHUNK, HID), jnp.float32),
            pltpu.VMEM((CHUNK, HID), jnp.float32),
            pltpu.SemaphoreType.DMA,
            pltpu.SemaphoreType.DMA,
            pltpu.SemaphoreType.DMA,
            pltpu.SemaphoreType.DMA,
        ],
    )


def _ln(x, g, bt):
    mu = jnp.mean(x, axis=-1, keepdims=True)
    xc = x - mu
    var = jnp.mean(xc * xc, axis=-1, keepdims=True)
    return xc * lax.rsqrt(var + EPS) * g + bt


def _tc_compute(f_ref, t_ref, addf_ref, addt_ref, g_ref, b_ref, o_ref):
    g = g_ref[0]
    bt = b_ref[0]
    y_f = _ln(f_ref[0] + addf_ref[...], g, bt)
    y_t = _ln(t_ref[0] + addt_ref[...], g, bt)
    o_ref[0, 0, :] = y_t[0]
    o_ref[0, pl.ds(1, LF), :] = y_f
    o_ref[0, pl.ds(LF + 1, LT - 1), :] = y_t[1:LT]


def _tc_body_a(f_ref, t_ref, addf_ref, addt_ref, g_ref, b_ref, o_ref):
    _tc_compute(f_ref, t_ref, addf_ref, addt_ref, g_ref, b_ref, o_ref)


def _tc_body_b(prev_ref, f_ref, t_ref, addf_ref, addt_ref, g_ref, b_ref, o_ref):
    del prev_ref
    _tc_compute(f_ref, t_ref, addf_ref, addt_ref, g_ref, b_ref, o_ref)


def kernel(text_input_ids, frame_inputs_embeds, past_key_values_length,
           word_emb, pos_emb, tok_emb, ln_gamma, ln_beta):
    ids_flat = text_input_ids.reshape(B * LT).astype(jnp.int32)
    gather = _make_sc_gather()
    stages = [
        gather(word_emb, ids_flat[s * NTOK_H:(s + 1) * NTOK_H]).reshape(BH, LT, HID)
        for s in range(NSTAGE)
    ]

    posr = lax.dynamic_slice_in_dim(pos_emb, past_key_values_length, SEQ)
    add_f = posr[1:LF + 1] + tok_emb[1]
    add_t = jnp.concatenate([
        posr[0:1] + tok_emb[1],
        posr[LF + 1:] + tok_emb[0],
    ], axis=0)
    g2 = ln_gamma.reshape(1, HID)
    b2 = ln_beta.reshape(1, HID)

    common_specs = [
        pl.BlockSpec((LF, HID), lambda b: (0, 0)),
        pl.BlockSpec((LT, HID), lambda b: (0, 0)),
        pl.BlockSpec((1, HID), lambda b: (0, 0)),
        pl.BlockSpec((1, HID), lambda b: (0, 0)),
    ]
    out_shape = jax.ShapeDtypeStruct((B, SEQ, HID), jnp.float32)

    acc = None
    for s in range(NSTAGE):
        fmap = lambda b, s=s: (b + s * BH, 0, 0)
        if s == 0:
            acc = pl.pallas_call(
                _tc_body_a,
                grid=(BH,),
                in_specs=[
                    pl.BlockSpec((1, LF, HID), fmap),
                    pl.BlockSpec((1, LT, HID), lambda b: (b, 0, 0)),
                    *common_specs,
                ],
                out_specs=pl.BlockSpec((1, SEQ, HID), fmap),
                out_shape=out_shape,
            )(frame_inputs_embeds, stages[s], add_f, add_t, g2, b2)
        else:
            acc = pl.pallas_call(
                _tc_body_b,
                grid=(BH,),
                in_specs=[
                    pl.BlockSpec(memory_space=pl.ANY),
                    pl.BlockSpec((1, LF, HID), fmap),
                    pl.BlockSpec((1, LT, HID), lambda b: (b, 0, 0)),
                    *common_specs,
                ],
                out_specs=pl.BlockSpec((1, SEQ, HID), fmap),
                out_shape=out_shape,
                input_output_aliases={0: 0},
            )(acc, frame_inputs_embeds, stages[s], add_f, add_t, g2, b2)
    return acc


# speedup vs baseline: 1.0070x; 1.0070x over previous
import functools

import jax
import jax.numpy as jnp
from jax import lax
from jax.experimental import pallas as pl
from jax.experimental.pallas import tpu as pltpu
from jax.experimental.pallas import tpu_sc as plsc

VOCAB = 100000
HID = 768
MAXPOS = 1024
EPS = 1e-12

NC = 2
NS = 16
NW = NC * NS

B = 32
LT = 512
LF = 512
SEQ = LT + LF
NTOK = B * LT
ROWS_PER_W = NTOK // NW
CHUNK = 64
NCHUNK = ROWS_PER_W // CHUNK


def _sc_gather_body(table_hbm, idx_hbm, out_hbm, idx_v, buf0, buf1,
                    sg0, sg1, sw0, sw1):
    wid = lax.axis_index("s") * NC + lax.axis_index("c")
    base = wid * ROWS_PER_W
    pltpu.sync_copy(idx_hbm.at[pl.ds(base, ROWS_PER_W)], idx_v)
    bufs = (buf0, buf1)
    gsems = (sg0, sg1)
    wsems = (sw0, sw1)
    gathers = [None, None]
    writes = [None, None]
    for c in range(NCHUNK):
        p = c % 2
        if writes[p] is not None:
            writes[p].wait()
        gathers[p] = pltpu.async_copy(
            table_hbm.at[idx_v.at[pl.ds(c * CHUNK, CHUNK)]], bufs[p], gsems[p]
        )
        if c >= 1:
            q = (c - 1) % 2
            gathers[q].wait()
            writes[q] = pltpu.async_copy(
                bufs[q], out_hbm.at[pl.ds(base + (c - 1) * CHUNK, CHUNK)],
                wsems[q],
            )
    p = (NCHUNK - 1) % 2
    gathers[p].wait()
    writes[p] = pltpu.async_copy(
        bufs[p], out_hbm.at[pl.ds(base + (NCHUNK - 1) * CHUNK, CHUNK)], wsems[p]
    )
    writes[p].wait()
    writes[1 - p].wait()


@functools.cache
def _make_sc_gather():
    return pl.kernel(
        _sc_gather_body,
        mesh=plsc.VectorSubcoreMesh(
            core_axis_name="c", subcore_axis_name="s",
            num_cores=NC, num_subcores=NS,
        ),
        out_type=jax.ShapeDtypeStruct((NTOK, HID), jnp.float32),
        scratch_types=[
            pltpu.VMEM((ROWS_PER_W,), jnp.int32),
            pltpu.VMEM((CHUNK, HID), jnp.float32),
            pltpu.VMEM((CHUNK, HID), jnp.float32),
            pltpu.SemaphoreType.DMA,
            pltpu.SemaphoreType.DMA,
            pltpu.SemaphoreType.DMA,
            pltpu.SemaphoreType.DMA,
        ],
    )


def _tc_body(f_ref, t_ref, addf_ref, addt_ref, g_ref, b_ref, o_ref):
    g = g_ref[0]
    bt = b_ref[0]

    def ln(x):
        mu = jnp.mean(x, axis=-1, keepdims=True)
        xc = x - mu
        var = jnp.mean(xc * xc, axis=-1, keepdims=True)
        return xc * lax.rsqrt(var + EPS) * g + bt

    y_f = ln(f_ref[0] + addf_ref[...])
    y_t = ln(t_ref[0] + addt_ref[...])
    o_ref[0, 0, :] = y_t[0]
    o_ref[0, pl.ds(1, LF), :] = y_f
    o_ref[0, pl.ds(LF + 1, LT - 1), :] = y_t[1:LT]


def kernel(text_input_ids, frame_inputs_embeds, past_key_values_length,
           word_emb, pos_emb, tok_emb, ln_gamma, ln_beta):
    ids_flat = text_input_ids.reshape(NTOK).astype(jnp.int32)
    tstage = _make_sc_gather()(word_emb, ids_flat).reshape(B, LT, HID)

    posr = lax.dynamic_slice_in_dim(pos_emb, past_key_values_length, SEQ)
    add_f = posr[1:LF + 1] + tok_emb[1]
    add_t = jnp.concatenate([
        posr[0:1] + tok_emb[1],
        posr[LF + 1:] + tok_emb[0],
    ], axis=0)

    out = pl.pallas_call(
        _tc_body,
        grid=(B,),
        in_specs=[
            pl.BlockSpec((1, LF, HID), lambda b: (b, 0, 0)),
            pl.BlockSpec((1, LT, HID), lambda b: (b, 0, 0)),
            pl.BlockSpec((LF, HID), lambda b: (0, 0)),
            pl.BlockSpec((LT, HID), lambda b: (0, 0)),
            pl.BlockSpec((1, HID), lambda b: (0, 0)),
            pl.BlockSpec((1, HID), lambda b: (0, 0)),
        ],
        out_specs=pl.BlockSpec((1, SEQ, HID), lambda b: (b, 0, 0)),
        out_shape=jax.ShapeDtypeStruct((B, SEQ, HID), jnp.float32),
    )(frame_inputs_embeds, tstage, add_f, add_t,
      ln_gamma.reshape(1, HID), ln_beta.reshape(1, HID))
    return out


# speedup vs baseline: 1.0671x; 1.0597x over previous
import functools

import jax
import jax.numpy as jnp
from jax import lax
from jax.experimental import pallas as pl
from jax.experimental.pallas import tpu as pltpu
from jax.experimental.pallas import tpu_sc as plsc

VOCAB = 100000
HID = 768
MAXPOS = 1024
EPS = 1e-12

NC = 2
NS = 16
NW = NC * NS

B = 32
LT = 512
LF = 512
SEQ = LT + LF
NTOK = B * LT
ROWS_PER_W = NTOK // NW
CHUNK = 64
NCHUNK = ROWS_PER_W // CHUNK


def _sc_gather_body(table_hbm, idx_hbm, out_hbm, idx_v, buf0, buf1,
                    sg0, sg1, sw0, sw1):
    wid = lax.axis_index("s") * NC + lax.axis_index("c")
    base = wid * ROWS_PER_W
    pltpu.sync_copy(idx_hbm.at[pl.ds(base, ROWS_PER_W)], idx_v)
    bufs = (buf0, buf1)
    gsems = (sg0, sg1)
    wsems = (sw0, sw1)
    gathers = [None, None]
    writes = [None, None]
    for c in range(NCHUNK):
        p = c % 2
        if writes[p] is not None:
            writes[p].wait()
        gathers[p] = pltpu.async_copy(
            table_hbm.at[idx_v.at[pl.ds(c * CHUNK, CHUNK)]], bufs[p], gsems[p]
        )
        if c >= 1:
            q = (c - 1) % 2
            gathers[q].wait()
            writes[q] = pltpu.async_copy(
                bufs[q], out_hbm.at[pl.ds(base + (c - 1) * CHUNK, CHUNK)],
                wsems[q],
            )
    p = (NCHUNK - 1) % 2
    gathers[p].wait()
    writes[p] = pltpu.async_copy(
        bufs[p], out_hbm.at[pl.ds(base + (NCHUNK - 1) * CHUNK, CHUNK)], wsems[p]
    )
    writes[p].wait()
    writes[1 - p].wait()


@functools.cache
def _make_sc_gather():
    return pl.kernel(
        _sc_gather_body,
        mesh=plsc.VectorSubcoreMesh(
            core_axis_name="c", subcore_axis_name="s",
            num_cores=NC, num_subcores=NS,
        ),
        out_type=jax.ShapeDtypeStruct((NTOK, HID), jnp.float32),
        scratch_types=[
            pltpu.VMEM((ROWS_PER_W,), jnp.int32),
            pltpu.VMEM((CHUNK, HID), jnp.float32),
            pltpu.VMEM((CHUNK, HID), jnp.float32),
            pltpu.SemaphoreType.DMA,
            pltpu.SemaphoreType.DMA,
            pltpu.SemaphoreType.DMA,
            pltpu.SemaphoreType.DMA,
        ],
    )


def _tc_body(f_ref, t_ref, addf_ref, addt_ref, g_ref, b_ref, o_ref):
    g = g_ref[0]
    bt = b_ref[0]

    def ln(x):
        mu = jnp.mean(x, axis=-1, keepdims=True)
        xc = x - mu
        var = jnp.mean(xc * xc, axis=-1, keepdims=True)
        return xc * lax.rsqrt(var + EPS) * g + bt

    for i in range(2):
        y_f = ln(f_ref[i] + addf_ref[...])
        y_t = ln(t_ref[i] + addt_ref[...])
        o_ref[i, 0, :] = y_t[0]
        o_ref[i, pl.ds(1, LF), :] = y_f
        o_ref[i, pl.ds(LF + 1, LT - 1), :] = y_t[1:LT]


def kernel(text_input_ids, frame_inputs_embeds, past_key_values_length,
           word_emb, pos_emb, tok_emb, ln_gamma, ln_beta):
    ids_flat = text_input_ids.reshape(NTOK).astype(jnp.int32)
    tstage = _make_sc_gather()(word_emb, ids_flat).reshape(B, LT, HID)

    posr = lax.dynamic_slice_in_dim(pos_emb, past_key_values_length, SEQ)
    add_f = posr[1:LF + 1] + tok_emb[1]
    add_t = jnp.concatenate([
        posr[0:1] + tok_emb[1],
        posr[LF + 1:] + tok_emb[0],
    ], axis=0)

    out = pl.pallas_call(
        _tc_body,
        grid=(B // 2,),
        in_specs=[
            pl.BlockSpec((2, LF, HID), lambda b: (b, 0, 0)),
            pl.BlockSpec((2, LT, HID), lambda b: (b, 0, 0)),
            pl.BlockSpec((LF, HID), lambda b: (0, 0)),
            pl.BlockSpec((LT, HID), lambda b: (0, 0)),
            pl.BlockSpec((1, HID), lambda b: (0, 0)),
            pl.BlockSpec((1, HID), lambda b: (0, 0)),
        ],
        out_specs=pl.BlockSpec((2, SEQ, HID), lambda b: (b, 0, 0)),
        out_shape=jax.ShapeDtypeStruct((B, SEQ, HID), jnp.float32),
    )(frame_inputs_embeds, tstage, add_f, add_t,
      ln_gamma.reshape(1, HID), ln_beta.reshape(1, HID))
    return out
